# baseline (speedup 1.0000x reference)
import functools

import jax
import jax.numpy as jnp
from jax import lax
from jax.experimental import pallas as pl
from jax.experimental.pallas import tpu as pltpu
from jax.experimental.pallas import tpu_sc as plsc

N_TOTAL = 16777216
N_BINS = 1024
NC = 2
NS = 16
L = 16
NW = NC * NS
PER_W = N_TOTAL // NW
BLK = 16384
NBLK = PER_W // BLK
NBUF = 4
NROUND = NBLK // NBUF

_mesh = plsc.VectorSubcoreMesh(core_axis_name="c", subcore_axis_name="s")


@functools.partial(
    pl.kernel,
    out_type=jax.ShapeDtypeStruct((NW, N_BINS), jnp.float32),
    mesh=_mesh,
    scratch_types=[
        pltpu.VMEM((BLK,), jnp.int32),
        pltpu.VMEM((BLK,), jnp.int32),
        pltpu.VMEM((BLK,), jnp.int32),
        pltpu.VMEM((BLK,), jnp.int32),
        pltpu.VMEM((L * N_BINS,), jnp.float32),
        pltpu.VMEM((N_BINS,), jnp.float32),
        pltpu.SemaphoreType.DMA,
        pltpu.SemaphoreType.DMA,
        pltpu.SemaphoreType.DMA,
        pltpu.SemaphoreType.DMA,
    ],
    compiler_params=pltpu.CompilerParams(needs_layout_passes=False),
)
def _sc_histogram(x_hbm, out_hbm, b0, b1, b2, b3, hists, hist1,
                  s0, s1, s2, s3):
    bufs = [b0, b1, b2, b3]
    sems = [s0, s1, s2, s3]
    wid = lax.axis_index("s") * NC + lax.axis_index("c")
    base = wid * PER_W

    for b in range(NBUF):
        pltpu.async_copy(x_hbm.at[pl.ds(base + b * BLK, BLK)], bufs[b], sems[b])

    zeros16 = jnp.zeros((L,), jnp.float32)

    @plsc.parallel_loop(0, (L * N_BINS) // L, unroll=8)
    def _zero(i):
        hists[pl.ds(i * L, L)] = zeros16

    lane_off = lax.iota(jnp.int32, L) * N_BINS
    ones16 = jnp.ones((L,), jnp.float32)

    def _scatter_block(buf):
        @plsc.parallel_loop(0, BLK // L, unroll=16)
        def _s(i):
            idx = buf[pl.ds(i * L, L)] + lane_off
            plsc.addupdate_scatter(hists, [idx], ones16)

    def _round(g, c):
        for b in range(NBUF):
            pltpu.make_async_copy(x_hbm.at[pl.ds(0, BLK)], bufs[b], sems[b]).wait()
            _scatter_block(bufs[b])

            @pl.when(g < NROUND - 1)
            def _():
                pltpu.async_copy(
                    x_hbm.at[pl.ds(base + ((g + 1) * NBUF + b) * BLK, BLK)],
                    bufs[b], sems[b])

        return c

    lax.fori_loop(0, NROUND, _round, 0)

    @plsc.parallel_loop(0, N_BINS // L, unroll=2)
    def _red(g):
        acc = hists[pl.ds(g * L, L)]
        for l in range(1, L):
            acc = acc + hists[pl.ds(l * N_BINS + g * L, L)]
        hist1[pl.ds(g * L, L)] = acc

    pltpu.sync_copy(hist1, out_hbm.at[wid])


def _entropy_body(counts_ref, out_ref):
    c = counts_ref[...]
    counts = jnp.sum(c, axis=0, keepdims=True)
    total = jnp.sum(counts)
    p = counts / total
    out_ref[0, 0] = -jnp.sum(p * jnp.log(p + 1e-08))


_entropy_tc = pl.pallas_call(
    _entropy_body,
    out_shape=jax.ShapeDtypeStruct((1, 1), jnp.float32),
    in_specs=[pl.BlockSpec(memory_space=pltpu.VMEM)],
    out_specs=pl.BlockSpec(memory_space=pltpu.SMEM),
)


def kernel(cluster_assignments, n_clusters):
    counts = _sc_histogram(cluster_assignments)
    return _entropy_tc(counts)[0, 0]


# speedup vs baseline: 1.3510x; 1.3510x over previous
import functools

import jax
import jax.numpy as jnp
from jax import lax
from jax.experimental import pallas as pl
from jax.experimental.pallas import tpu as pltpu
from jax.experimental.pallas import tpu_sc as plsc

N_TOTAL = 16777216
N_BINS = 1024
NC = 2
NS = 16
L = 16
NW = NC * NS
PER_W = N_TOTAL // NW
BLK = 16384
NBLK = PER_W // BLK
NBUF = 4
NROUND = NBLK // NBUF

_mesh = plsc.VectorSubcoreMesh(core_axis_name="c", subcore_axis_name="s")


@functools.partial(
    pl.kernel,
    out_type=jax.ShapeDtypeStruct((NW, N_BINS), jnp.float32),
    mesh=_mesh,
    scratch_types=[
        pltpu.VMEM((BLK,), jnp.int32),
        pltpu.VMEM((BLK,), jnp.int32),
        pltpu.VMEM((BLK,), jnp.int32),
        pltpu.VMEM((BLK,), jnp.int32),
        pltpu.VMEM((L * N_BINS,), jnp.float32),
        pltpu.VMEM((N_BINS,), jnp.float32),
        pltpu.SemaphoreType.DMA,
        pltpu.SemaphoreType.DMA,
        pltpu.SemaphoreType.DMA,
        pltpu.SemaphoreType.DMA,
    ],
    compiler_params=pltpu.CompilerParams(needs_layout_passes=False),
)
def _sc_histogram(x_hbm, out_hbm, b0, b1, b2, b3, hists, hist1,
                  s0, s1, s2, s3):
    bufs = [b0, b1, b2, b3]
    sems = [s0, s1, s2, s3]
    wid = lax.axis_index("s") * NC + lax.axis_index("c")
    base = wid * PER_W

    for b in range(NBUF):
        pltpu.async_copy(x_hbm.at[pl.ds(base + b * BLK, BLK)], bufs[b], sems[b])

    zeros16 = jnp.zeros((L,), jnp.float32)

    @plsc.parallel_loop(0, (L * N_BINS) // L, unroll=8)
    def _zero(i):
        hists[pl.ds(i * L, L)] = zeros16

    lane_off = lax.iota(jnp.int32, L)
    ones16 = jnp.ones((L,), jnp.float32)

    def _scatter_block(buf):
        @plsc.parallel_loop(0, BLK // L, unroll=16)
        def _s(i):
            idx = lax.shift_left(buf[pl.ds(i * L, L)], 4) + lane_off
            plsc.addupdate_scatter(hists, [idx], ones16)

    def _round(g, c):
        for b in range(NBUF):
            pltpu.make_async_copy(x_hbm.at[pl.ds(0, BLK)], bufs[b], sems[b]).wait()
            _scatter_block(bufs[b])

            @pl.when(g < NROUND - 1)
            def _():
                pltpu.async_copy(
                    x_hbm.at[pl.ds(base + ((g + 1) * NBUF + b) * BLK, BLK)],
                    bufs[b], sems[b])

        return c

    lax.fori_loop(0, NROUND, _round, 0)

    red_idx = lax.iota(jnp.int32, L) * L

    @plsc.parallel_loop(0, N_BINS // L, unroll=2)
    def _red(g):
        base_i = g * (L * L) + red_idx
        acc = plsc.load_gather(hists, [base_i])
        for j in range(1, L):
            acc = acc + plsc.load_gather(hists, [base_i + j])
        hist1[pl.ds(g * L, L)] = acc

    pltpu.sync_copy(hist1, out_hbm.at[wid])


def _entropy_body(counts_ref, out_ref):
    c = counts_ref[...]
    counts = jnp.sum(c, axis=0, keepdims=True)
    total = jnp.sum(counts)
    p = counts / total
    out_ref[0, 0] = -jnp.sum(p * jnp.log(p + 1e-08))


_entropy_tc = pl.pallas_call(
    _entropy_body,
    out_shape=jax.ShapeDtypeStruct((1, 1), jnp.float32),
    in_specs=[pl.BlockSpec(memory_space=pltpu.VMEM)],
    out_specs=pl.BlockSpec(memory_space=pltpu.SMEM),
)


def kernel(cluster_assignments, n_clusters):
    counts = _sc_histogram(cluster_assignments)
    return _entropy_tc(counts)[0, 0]
